# Initial kernel scaffold; baseline (speedup 1.0000x reference)
import jax
import jax.numpy as jnp
from jax.experimental import pallas as pl


def kernel(token_ids, token_table, pos_table):
    raise NotImplementedError("write your pallas kernel here")


# speedup vs baseline: 1.6148x; 1.6148x over previous
import functools

import jax
import jax.numpy as jnp
from jax import lax
from jax.experimental import pallas as pl
from jax.experimental.pallas import tpu as pltpu
from jax.experimental.pallas import tpu_sc as plsc

VOCAB = 100000
DIM = 128
B = 4096
S = 200

NC = 2
NS = 16
NW = NC * NS

CHUNK = 100
TOK = B * S
NROW = TOK // CHUNK
ROWS_PER_W = NROW // NW


def _sc_body(ids_hbm, table_hbm, pos_hbm, out_hbm, idx_v, pos_v, buf, gsem):
    wid = lax.axis_index("s") * NC + lax.axis_index("c")
    row0 = wid * ROWS_PER_W

    pltpu.sync_copy(ids_hbm.at[pl.ds(row0, ROWS_PER_W)], idx_v)
    pltpu.sync_copy(pos_hbm, pos_v)

    def chunk_body(c, _):
        pltpu.async_copy(table_hbm.at[idx_v.at[c]], buf, gsem).wait()
        half = lax.rem(c, 2)

        def add_body(r, _):
            for j in range(DIM // 16):
                sl = pl.ds(j * 16, 16)
                buf[r, sl] = buf[r, sl] + pos_v[half, r, sl]
            return ()

        lax.fori_loop(0, CHUNK, add_body, (), unroll=2)
        pltpu.sync_copy(buf, out_hbm.at[row0 + c])
        return ()

    lax.fori_loop(0, ROWS_PER_W, chunk_body, ())


@functools.partial(jax.jit, static_argnames=())
def kernel(token_ids, token_table, pos_table):
    ids = token_ids.astype(jnp.int32).reshape(NROW, CHUNK)
    pos = pos_table.astype(jnp.float32).reshape(S // CHUNK, CHUNK, DIM)

    mesh = plsc.VectorSubcoreMesh(
        core_axis_name="c", subcore_axis_name="s", num_cores=NC,
        num_subcores=NS)
    out = pl.kernel(
        _sc_body,
        out_type=jax.ShapeDtypeStruct((NROW, CHUNK, DIM), jnp.float32),
        mesh=mesh,
        scratch_types=[
            pltpu.VMEM((ROWS_PER_W, CHUNK), jnp.int32),
            pltpu.VMEM((S // CHUNK, CHUNK, DIM), jnp.float32),
            pltpu.VMEM((CHUNK, DIM), jnp.float32),
            pltpu.SemaphoreType.DMA,
        ],
    )(ids, token_table, pos)
    return out.reshape(B, S, DIM)


# speedup vs baseline: 1.9921x; 1.2337x over previous
import functools

import jax
import jax.numpy as jnp
from jax import lax
from jax.experimental import pallas as pl
from jax.experimental.pallas import tpu as pltpu
from jax.experimental.pallas import tpu_sc as plsc

VOCAB = 100000
DIM = 128
B = 4096
S = 200

NC = 2
NS = 16
NW = NC * NS

CHUNK = 100
TOK = B * S
NROW = TOK // CHUNK
ROWS_PER_W = NROW // NW
NBUF = 4


def _sc_body(ids_hbm, table_hbm, pos_hbm, out_hbm, idx_v, pos_v, bufs, gsems, osems):
    wid = lax.axis_index("s") * NC + lax.axis_index("c")
    row0 = wid * ROWS_PER_W

    pltpu.sync_copy(ids_hbm.at[pl.ds(row0, ROWS_PER_W)], idx_v)
    pltpu.sync_copy(pos_hbm, pos_v)

    def start_gather(c, b):
        pltpu.async_copy(table_hbm.at[idx_v.at[c]], bufs[b], gsems[b])

    def wait_gather(c, b):
        pltpu.make_async_copy(table_hbm.at[idx_v.at[c]], bufs[b], gsems[b]).wait()

    def start_out(c, b):
        pltpu.async_copy(bufs[b], out_hbm.at[row0 + c], osems[b])

    def wait_out(c, b):
        pltpu.make_async_copy(bufs[b], out_hbm.at[row0 + c], osems[b]).wait()

    for b in range(NBUF - 1):
        start_gather(b, b)

    def group_body(g, _):
        for b in range(NBUF):
            c = g * NBUF + b
            bn = (b + NBUF - 1) % NBUF

            @pl.when(c >= 1)
            def _wait_prev_out():
                wait_out(c - 1, bn)

            @pl.when(c + NBUF - 1 < ROWS_PER_W)
            def _fire():
                start_gather(c + NBUF - 1, bn)

            wait_gather(c, b)
            half = lax.rem(c, 2)
            buf = bufs[b]

            def add_body(r, _):
                for j in range(DIM // 16):
                    sl = pl.ds(j * 16, 16)
                    buf[r, sl] = buf[r, sl] + pos_v[half, r, sl]
                return ()

            lax.fori_loop(0, CHUNK, add_body, (), unroll=4)
            start_out(c, b)
        return ()

    lax.fori_loop(0, ROWS_PER_W // NBUF, group_body, ())

    wait_out(ROWS_PER_W - 1, NBUF - 1)


@functools.partial(jax.jit, static_argnames=())
def kernel(token_ids, token_table, pos_table):
    ids = token_ids.astype(jnp.int32).reshape(NROW, CHUNK)
    pos = pos_table.astype(jnp.float32).reshape(S // CHUNK, CHUNK, DIM)

    mesh = plsc.VectorSubcoreMesh(
        core_axis_name="c", subcore_axis_name="s", num_cores=NC,
        num_subcores=NS)
    out = pl.kernel(
        _sc_body,
        out_type=jax.ShapeDtypeStruct((NROW, CHUNK, DIM), jnp.float32),
        mesh=mesh,
        scratch_types=[
            pltpu.VMEM((ROWS_PER_W, CHUNK), jnp.int32),
            pltpu.VMEM((S // CHUNK, CHUNK, DIM), jnp.float32),
            [pltpu.VMEM((CHUNK, DIM), jnp.float32) for _ in range(NBUF)],
            [pltpu.SemaphoreType.DMA for _ in range(NBUF)],
            [pltpu.SemaphoreType.DMA for _ in range(NBUF)],
        ],
    )(ids, token_table, pos)
    return out.reshape(B, S, DIM)


# speedup vs baseline: 4.2170x; 2.1168x over previous
import functools

import jax
import jax.numpy as jnp
from jax import lax
from jax.experimental import pallas as pl
from jax.experimental.pallas import tpu as pltpu
from jax.experimental.pallas import tpu_sc as plsc

VOCAB = 100000
DIM = 128
B = 4096
S = 200

NC = 2
NS = 16
NW = NC * NS

CHUNK = 100
TOK = B * S
NROW = TOK // CHUNK
ROWS_PER_W = NROW // NW
NBUF = 4


def _sc_body(ids_hbm, table_hbm, pos_hbm, out_hbm, idx_v, pos_v, bufs, gsems, osems):
    wid = lax.axis_index("s") * NC + lax.axis_index("c")
    row0 = wid * ROWS_PER_W

    pltpu.sync_copy(ids_hbm.at[pl.ds(row0, ROWS_PER_W)], idx_v)
    pltpu.sync_copy(pos_hbm, pos_v)

    def start_gather(c, b):
        pltpu.async_copy(table_hbm.at[idx_v.at[c]], bufs[b], gsems[b])

    def wait_gather(c, b):
        pltpu.make_async_copy(table_hbm.at[idx_v.at[c]], bufs[b], gsems[b]).wait()

    def start_out(c, b):
        pltpu.async_copy(bufs[b], out_hbm.at[row0 + c], osems[b])

    def wait_out(c, b):
        pltpu.make_async_copy(bufs[b], out_hbm.at[row0 + c], osems[b]).wait()

    for b in range(NBUF - 1):
        start_gather(b, b)

    def group_body(g, _):
        for b in range(NBUF):
            c = g * NBUF + b
            bn = (b + NBUF - 1) % NBUF

            wait_gather(c, b)
            half = lax.rem(c, 2)
            buf = bufs[b]

            @functools.partial(plsc.parallel_loop, 0, CHUNK, unroll=4)
            def _add(r):
                for j in range(DIM // 16):
                    sl = pl.ds(j * 16, 16)
                    plsc.addupdate(buf.at[r, sl], pos_v[half, r, sl])

            start_out(c, b)

            @pl.when(c >= 1)
            def _wait_prev_out():
                wait_out(c - 1, bn)

            @pl.when(c + NBUF - 1 < ROWS_PER_W)
            def _fire():
                start_gather(c + NBUF - 1, bn)
        return ()

    lax.fori_loop(0, ROWS_PER_W // NBUF, group_body, ())

    wait_out(ROWS_PER_W - 1, NBUF - 1)


@functools.partial(jax.jit, static_argnames=())
def kernel(token_ids, token_table, pos_table):
    ids = token_ids.astype(jnp.int32).reshape(NROW, CHUNK)
    pos = pos_table.astype(jnp.float32).reshape(S // CHUNK, CHUNK, DIM)

    mesh = plsc.VectorSubcoreMesh(
        core_axis_name="c", subcore_axis_name="s", num_cores=NC,
        num_subcores=NS)
    out = pl.kernel(
        _sc_body,
        out_type=jax.ShapeDtypeStruct((NROW, CHUNK, DIM), jnp.float32),
        mesh=mesh,
        scratch_types=[
            pltpu.VMEM((ROWS_PER_W, CHUNK), jnp.int32),
            pltpu.VMEM((S // CHUNK, CHUNK, DIM), jnp.float32),
            [pltpu.VMEM((CHUNK, DIM), jnp.float32) for _ in range(NBUF)],
            [pltpu.SemaphoreType.DMA for _ in range(NBUF)],
            [pltpu.SemaphoreType.DMA for _ in range(NBUF)],
        ],
    )(ids, token_table, pos)
    return out.reshape(B, S, DIM)
